# baseline (speedup 1.0000x reference)
import functools
import jax
import jax.numpy as jnp
from jax.experimental import pallas as pl
from jax.experimental.pallas import tpu as pltpu

_NPOINTS = [1024, 256, 64, 16]
_NSAMPLES = [32, 32, 32, 32]


def _fc_head_kernel(x_ref, w1_ref, b1_ref, w2_ref, b2_ref, o_ref):
    h = jnp.maximum(x_ref[...] @ w1_ref[...] + b1_ref[...], 0.0)
    o_ref[...] = h @ w2_ref[...] + b2_ref[...]


def _fc_head(x, fc_params):
    (w1, b1), (w2, b2) = fc_params
    B, N, C = x.shape
    C2 = w2.shape[1]
    C2p = 128
    w2p = jnp.zeros((w2.shape[0], C2p), w2.dtype).at[:, :C2].set(w2)
    b2p = jnp.zeros((C2p,), b2.dtype).at[:C2].set(b2)
    xf = x.reshape(B * N, C)
    out = pl.pallas_call(
        _fc_head_kernel,
        out_shape=jax.ShapeDtypeStruct((B * N, C2p), jnp.float32),
        grid=(B * N // 2048,),
        in_specs=[
            pl.BlockSpec((2048, C), lambda i: (i, 0)),
            pl.BlockSpec((C, C), lambda i: (0, 0)),
            pl.BlockSpec((C,), lambda i: (0,)),
            pl.BlockSpec((C, C2p), lambda i: (0, 0)),
            pl.BlockSpec((C2p,), lambda i: (0,)),
        ],
        out_specs=pl.BlockSpec((2048, C2p), lambda i: (i, 0)),
    )(xf, w1, b1, w2p, b2p)
    return out.reshape(B, N, C2p)[:, :, :C2]


def _mlp(h, layers, last_act=True):
    n = len(layers)
    for i, (W, b) in enumerate(layers):
        h = h @ W + b
        if last_act or i < n - 1:
            h = jax.nn.relu(h)
    return h


def _farthest(xyz, npoint):
    Bn, Nn, _ = xyz.shape

    def step(carry, _):
        dist, far = carry
        centroid = xyz[jnp.arange(Bn), far]
        d = jnp.sum((xyz - centroid[:, None, :]) ** 2, -1)
        dist = jnp.minimum(dist, d)
        nxt = jnp.argmax(dist, -1).astype(jnp.int32)
        return (dist, nxt), far

    dist0 = jnp.full((Bn, Nn), 1e10, dtype=jnp.float32)
    far0 = jnp.zeros((Bn,), dtype=jnp.int32)
    _, idx = jax.lax.scan(step, (dist0, far0), None, length=npoint)
    return jnp.transpose(idx, (1, 0))


def _take_rows(x, idx):
    bi = jnp.arange(x.shape[0]).reshape((x.shape[0],) + (1,) * (idx.ndim - 1))
    return x[bi, idx]


def _nn_idx(ref, query, k):
    d = jnp.sum((query[:, :, None, :] - ref[:, None, :, :]) ** 2, -1)
    _, idx = jax.lax.top_k(-d, k)
    return idx


def _sa_stage(xyz, feat, npoint, nsample, layers):
    fps_idx = _farthest(xyz, npoint)
    new_xyz = _take_rows(xyz, fps_idx)
    nn_idx = _nn_idx(xyz, new_xyz, nsample)
    g_xyz = _take_rows(xyz, nn_idx) - new_xyz[:, :, None, :]
    g_feat = _take_rows(feat, nn_idx)
    h = jnp.concatenate([g_xyz, g_feat], -1)
    h = _mlp(h, layers)
    return new_xyz, jnp.max(h, axis=2)


def _fp_stage(xyz1, xyz2, feat1, feat2, layers):
    d = jnp.sum((xyz1[:, :, None, :] - xyz2[:, None, :, :]) ** 2, -1)
    negd, idx = jax.lax.top_k(-d, 3)
    dist = jnp.maximum(-negd, 1e-10)
    w = 1.0 / dist
    w = w / jnp.sum(w, -1, keepdims=True)
    interp = jnp.sum(_take_rows(feat2, idx) * w[..., None], axis=2)
    h = jnp.concatenate([interp, feat1], -1)
    return _mlp(h, layers)


def kernel(pointcloud, params):
    xyz = pointcloud[..., :3]
    feat = pointcloud[..., 3:]
    l_xyz = [xyz]
    l_feat = [feat]
    for i in range(4):
        nx, nf = _sa_stage(l_xyz[i], l_feat[i], _NPOINTS[i], _NSAMPLES[i], params["sa"][i])
        l_xyz.append(nx)
        l_feat.append(nf)
    for i in range(-1, -5, -1):
        l_feat[i - 1] = _fp_stage(l_xyz[i - 1], l_xyz[i], l_feat[i - 1], l_feat[i], params["fp"][i])
    h = _fc_head(l_feat[0], params["fc"])
    return jnp.transpose(h, (0, 2, 1))


# speedup vs baseline: 1.6200x; 1.6200x over previous
import functools
import jax
import jax.numpy as jnp
from jax.experimental import pallas as pl
from jax.experimental.pallas import tpu as pltpu

_NPOINTS = [1024, 256, 64, 16]
_NSAMPLES = [32, 32, 32, 32]


def _fc_head_kernel(x_ref, w1_ref, b1_ref, w2_ref, b2_ref, o_ref):
    h = jnp.maximum(x_ref[...] @ w1_ref[...] + b1_ref[...], 0.0)
    o_ref[...] = h @ w2_ref[...] + b2_ref[...]


def _fc_head(x, fc_params):
    (w1, b1), (w2, b2) = fc_params
    B, N, C = x.shape
    C2 = w2.shape[1]
    C2p = 128
    w2p = jnp.zeros((w2.shape[0], C2p), w2.dtype).at[:, :C2].set(w2)
    b2p = jnp.zeros((C2p,), b2.dtype).at[:C2].set(b2)
    xf = x.reshape(B * N, C)
    out = pl.pallas_call(
        _fc_head_kernel,
        out_shape=jax.ShapeDtypeStruct((B * N, C2p), jnp.float32),
        grid=(B * N // 2048,),
        in_specs=[
            pl.BlockSpec((2048, C), lambda i: (i, 0)),
            pl.BlockSpec((C, C), lambda i: (0, 0)),
            pl.BlockSpec((C,), lambda i: (0,)),
            pl.BlockSpec((C, C2p), lambda i: (0, 0)),
            pl.BlockSpec((C2p,), lambda i: (0,)),
        ],
        out_specs=pl.BlockSpec((2048, C2p), lambda i: (i, 0)),
    )(xf, w1, b1, w2p, b2p)
    return out.reshape(B, N, C2p)[:, :, :C2]


_SENT = 1e9


def _fps_level(src, valid_n, npoint, dst_ref, b):
    S = src[0].shape[0]
    Sp = dst_ref.shape[2]
    fi = (jax.lax.broadcasted_iota(jnp.int32, (S, 128), 0) * 128
          + jax.lax.broadcasted_iota(jnp.int32, (S, 128), 1))
    fq = (jax.lax.broadcasted_iota(jnp.int32, (Sp, 128), 0) * 128
          + jax.lax.broadcasted_iota(jnp.int32, (Sp, 128), 1))
    x, y, z = src
    dist0 = jnp.where(fi < valid_n, 1e10, -1.0).astype(jnp.float32)

    def body(step, carry):
        dist, far = carry
        m = fi == far
        cx = jnp.sum(jnp.where(m, x, 0.0))
        cy = jnp.sum(jnp.where(m, y, 0.0))
        cz = jnp.sum(jnp.where(m, z, 0.0))
        mq = fq == step
        dst_ref[b, 0] = jnp.where(mq, cx, dst_ref[b, 0])
        dst_ref[b, 1] = jnp.where(mq, cy, dst_ref[b, 1])
        dst_ref[b, 2] = jnp.where(mq, cz, dst_ref[b, 2])
        d = (x - cx) ** 2 + (y - cy) ** 2 + (z - cz) ** 2
        dist = jnp.minimum(dist, d)
        mx = jnp.max(dist)
        far = jnp.min(jnp.where(dist == mx, fi, jnp.int32(1 << 30)))
        return dist, far

    jax.lax.fori_loop(0, npoint, body, (dist0, jnp.int32(0)), unroll=False)


def _fps_all_kernel(xyz_ref, nx1_ref, nx2_ref, nx3_ref, nx4_ref):
    nx1_ref[...] = jnp.zeros(nx1_ref.shape, jnp.float32)
    nx2_ref[...] = jnp.zeros(nx2_ref.shape, jnp.float32)
    nx3_ref[...] = jnp.full(nx3_ref.shape, _SENT, jnp.float32)
    nx4_ref[...] = jnp.full(nx4_ref.shape, _SENT, jnp.float32)
    for b in range(xyz_ref.shape[0]):
        src0 = [xyz_ref[b, c] for c in range(3)]
        _fps_level(src0, 8192, 1024, nx1_ref, b)
        src1 = [nx1_ref[b, c] for c in range(3)]
        _fps_level(src1, 1024, 256, nx2_ref, b)
        src2 = [nx2_ref[b, c] for c in range(3)]
        _fps_level(src2, 256, 64, nx3_ref, b)
        src3 = [nx3_ref[b, c] for c in range(3)]
        _fps_level(src3, 64, 16, nx4_ref, b)


def _fps_all(xyz):
    Bn, N, _ = xyz.shape
    xp = jnp.transpose(xyz, (0, 2, 1)).reshape(Bn, 3, N // 128, 128)
    outs = pl.pallas_call(
        _fps_all_kernel,
        out_shape=[
            jax.ShapeDtypeStruct((Bn, 3, 8, 128), jnp.float32),
            jax.ShapeDtypeStruct((Bn, 3, 2, 128), jnp.float32),
            jax.ShapeDtypeStruct((Bn, 3, 1, 128), jnp.float32),
            jax.ShapeDtypeStruct((Bn, 3, 1, 128), jnp.float32),
        ],
    )(xp)
    rows = []
    for p, npoint in zip(outs, _NPOINTS):
        rows.append(jnp.transpose(p.reshape(Bn, 3, -1), (0, 2, 1))[:, :npoint])
    return rows


def _mlp(h, layers, last_act=True):
    n = len(layers)
    for i, (W, b) in enumerate(layers):
        h = h @ W + b
        if last_act or i < n - 1:
            h = jax.nn.relu(h)
    return h


def _farthest(xyz, npoint):
    Bn, Nn, _ = xyz.shape

    def step(carry, _):
        dist, far = carry
        centroid = xyz[jnp.arange(Bn), far]
        d = jnp.sum((xyz - centroid[:, None, :]) ** 2, -1)
        dist = jnp.minimum(dist, d)
        nxt = jnp.argmax(dist, -1).astype(jnp.int32)
        return (dist, nxt), far

    dist0 = jnp.full((Bn, Nn), 1e10, dtype=jnp.float32)
    far0 = jnp.zeros((Bn,), dtype=jnp.int32)
    _, idx = jax.lax.scan(step, (dist0, far0), None, length=npoint)
    return jnp.transpose(idx, (1, 0))


def _take_rows(x, idx):
    bi = jnp.arange(x.shape[0]).reshape((x.shape[0],) + (1,) * (idx.ndim - 1))
    return x[bi, idx]


def _nn_idx(ref, query, k):
    d = jnp.sum((query[:, :, None, :] - ref[:, None, :, :]) ** 2, -1)
    _, idx = jax.lax.top_k(-d, k)
    return idx


def _sa_stage(xyz, feat, new_xyz, nsample, layers):
    nn_idx = _nn_idx(xyz, new_xyz, nsample)
    g_xyz = _take_rows(xyz, nn_idx) - new_xyz[:, :, None, :]
    g_feat = _take_rows(feat, nn_idx)
    h = jnp.concatenate([g_xyz, g_feat], -1)
    h = _mlp(h, layers)
    return new_xyz, jnp.max(h, axis=2)


def _fp_stage(xyz1, xyz2, feat1, feat2, layers):
    d = jnp.sum((xyz1[:, :, None, :] - xyz2[:, None, :, :]) ** 2, -1)
    negd, idx = jax.lax.top_k(-d, 3)
    dist = jnp.maximum(-negd, 1e-10)
    w = 1.0 / dist
    w = w / jnp.sum(w, -1, keepdims=True)
    interp = jnp.sum(_take_rows(feat2, idx) * w[..., None], axis=2)
    h = jnp.concatenate([interp, feat1], -1)
    return _mlp(h, layers)


def kernel(pointcloud, params):
    xyz = pointcloud[..., :3]
    feat = pointcloud[..., 3:]
    new_xyzs = _fps_all(xyz)
    l_xyz = [xyz]
    l_feat = [feat]
    for i in range(4):
        nx = new_xyzs[i]
        _, nf = _sa_stage(l_xyz[i], l_feat[i], nx, _NSAMPLES[i], params["sa"][i])
        l_xyz.append(nx)
        l_feat.append(nf)
    for i in range(-1, -5, -1):
        l_feat[i - 1] = _fp_stage(l_xyz[i - 1], l_xyz[i], l_feat[i - 1], l_feat[i], params["fp"][i])
    h = _fc_head(l_feat[0], params["fc"])
    return jnp.transpose(h, (0, 2, 1))


# speedup vs baseline: 3.8329x; 2.3660x over previous
import functools
import jax
import jax.numpy as jnp
from jax.experimental import pallas as pl
from jax.experimental.pallas import tpu as pltpu

_NPOINTS = [1024, 256, 64, 16]
_NSAMPLES = [32, 32, 32, 32]


def _fc_head_kernel(x_ref, w1_ref, b1_ref, w2_ref, b2_ref, o_ref):
    h = jnp.maximum(x_ref[...] @ w1_ref[...] + b1_ref[...], 0.0)
    o_ref[...] = h @ w2_ref[...] + b2_ref[...]


def _fc_head(x, fc_params):
    (w1, b1), (w2, b2) = fc_params
    B, N, C = x.shape
    C2 = w2.shape[1]
    C2p = 128
    w2p = jnp.zeros((w2.shape[0], C2p), w2.dtype).at[:, :C2].set(w2)
    b2p = jnp.zeros((C2p,), b2.dtype).at[:C2].set(b2)
    xf = x.reshape(B * N, C)
    out = pl.pallas_call(
        _fc_head_kernel,
        out_shape=jax.ShapeDtypeStruct((B * N, C2p), jnp.float32),
        grid=(B * N // 2048,),
        in_specs=[
            pl.BlockSpec((2048, C), lambda i: (i, 0)),
            pl.BlockSpec((C, C), lambda i: (0, 0)),
            pl.BlockSpec((C,), lambda i: (0,)),
            pl.BlockSpec((C, C2p), lambda i: (0, 0)),
            pl.BlockSpec((C2p,), lambda i: (0,)),
        ],
        out_specs=pl.BlockSpec((2048, C2p), lambda i: (i, 0)),
    )(xf, w1, b1, w2p, b2p)
    return out.reshape(B, N, C2p)[:, :, :C2]


_SENT = 1e9


def _fps_level(src, valid_n, npoint, dst_ref, b):
    S = src[0].shape[0]
    Sp = dst_ref.shape[2]
    fi = (jax.lax.broadcasted_iota(jnp.int32, (S, 128), 0) * 128
          + jax.lax.broadcasted_iota(jnp.int32, (S, 128), 1))
    fq = (jax.lax.broadcasted_iota(jnp.int32, (Sp, 128), 0) * 128
          + jax.lax.broadcasted_iota(jnp.int32, (Sp, 128), 1))
    x, y, z = src
    dist0 = jnp.where(fi < valid_n, 1e10, -1.0).astype(jnp.float32)

    def body(step, carry):
        dist, far = carry
        m = fi == far
        cx = jnp.sum(jnp.where(m, x, 0.0))
        cy = jnp.sum(jnp.where(m, y, 0.0))
        cz = jnp.sum(jnp.where(m, z, 0.0))
        mq = fq == step
        dst_ref[b, 0] = jnp.where(mq, cx, dst_ref[b, 0])
        dst_ref[b, 1] = jnp.where(mq, cy, dst_ref[b, 1])
        dst_ref[b, 2] = jnp.where(mq, cz, dst_ref[b, 2])
        d = (x - cx) ** 2 + (y - cy) ** 2 + (z - cz) ** 2
        dist = jnp.minimum(dist, d)
        mx = jnp.max(dist)
        far = jnp.min(jnp.where(dist == mx, fi, jnp.int32(1 << 30)))
        return dist, far

    jax.lax.fori_loop(0, npoint, body, (dist0, jnp.int32(0)), unroll=False)


def _fps_all_kernel(xyz_ref, nx1_ref, nx2_ref, nx3_ref, nx4_ref):
    nx1_ref[...] = jnp.zeros(nx1_ref.shape, jnp.float32)
    nx2_ref[...] = jnp.zeros(nx2_ref.shape, jnp.float32)
    nx3_ref[...] = jnp.full(nx3_ref.shape, _SENT, jnp.float32)
    nx4_ref[...] = jnp.full(nx4_ref.shape, _SENT, jnp.float32)
    for b in range(xyz_ref.shape[0]):
        src0 = [xyz_ref[b, c] for c in range(3)]
        _fps_level(src0, 8192, 1024, nx1_ref, b)
        src1 = [nx1_ref[b, c] for c in range(3)]
        _fps_level(src1, 1024, 256, nx2_ref, b)
        src2 = [nx2_ref[b, c] for c in range(3)]
        _fps_level(src2, 256, 64, nx3_ref, b)
        src3 = [nx3_ref[b, c] for c in range(3)]
        _fps_level(src3, 64, 16, nx4_ref, b)


def _fps_all_planes(xyz):
    Bn, N, _ = xyz.shape
    xp = jnp.transpose(xyz, (0, 2, 1)).reshape(Bn, 3, N // 128, 128)
    return pl.pallas_call(
        _fps_all_kernel,
        out_shape=[
            jax.ShapeDtypeStruct((Bn, 3, 8, 128), jnp.float32),
            jax.ShapeDtypeStruct((Bn, 3, 2, 128), jnp.float32),
            jax.ShapeDtypeStruct((Bn, 3, 1, 128), jnp.float32),
            jax.ShapeDtypeStruct((Bn, 3, 1, 128), jnp.float32),
        ],
    )(xp)


def _take_rows(x, idx):
    bi = jnp.arange(x.shape[0]).reshape((x.shape[0],) + (1,) * (idx.ndim - 1))
    return x[bi, idx]


def _knn_kernel(ref_ref, q_ref, o_ref, *, k):
    N = ref_ref.shape[1]
    rx = ref_ref[0, :, 0:1]
    ry = ref_ref[0, :, 1:2]
    rz = ref_ref[0, :, 2:3]
    qx = q_ref[0, 0, 0, :].reshape(1, 128)
    qy = q_ref[0, 0, 1, :].reshape(1, 128)
    qz = q_ref[0, 0, 2, :].reshape(1, 128)
    d = (qx - rx) ** 2 + (qy - ry) ** 2 + (qz - rz) ** 2
    ri = jax.lax.broadcasted_iota(jnp.int32, (N, 128), 0)
    big_i = jnp.int32(1 << 30)
    for j in range(k):
        m = jnp.min(d, axis=0, keepdims=True)
        idx = jnp.min(jnp.where(d == m, ri, big_i), axis=0, keepdims=True)
        o_ref[0, 0, j, :] = idx.reshape(128)
        d = jnp.where(ri == idx, jnp.float32(jnp.inf), d)


def _knn(ref_rows, q_planes, k):
    B, N, _ = ref_rows.shape
    QB = q_planes.shape[1]
    out = pl.pallas_call(
        functools.partial(_knn_kernel, k=k),
        out_shape=jax.ShapeDtypeStruct((B, QB, k, 128), jnp.int32),
        grid=(B, QB),
        in_specs=[
            pl.BlockSpec((1, N, 3), lambda b, q: (b, 0, 0)),
            pl.BlockSpec((1, 1, 3, 128), lambda b, q: (b, q, 0, 0)),
        ],
        out_specs=pl.BlockSpec((1, 1, k, 128), lambda b, q: (b, q, 0, 0)),
    )(ref_rows, q_planes)
    return jnp.transpose(out, (0, 2, 1, 3)).reshape(B, k, QB * 128)


def _sa_mlp_kernel(g_ref, *rest, nsample):
    w_refs = rest[:-1]
    o_ref = rest[-1]
    ns, QG, Cin = g_ref.shape[1], g_ref.shape[2], g_ref.shape[3]
    h = g_ref[0].reshape(ns * QG, Cin)
    nl = len(w_refs) // 2
    for i in range(nl):
        W = w_refs[2 * i][...]
        b = w_refs[2 * i + 1][...]
        h = jnp.maximum(jnp.dot(h, W, precision=jax.lax.Precision.HIGHEST) + b, 0.0)
    Cout = h.shape[1]
    h = h.reshape(ns, QG, Cout)
    acc = h[0]
    for s in range(1, ns):
        acc = jnp.maximum(acc, h[s])
    o_ref[0] = acc


def _sa_mlp_max(g, layers):
    B, ns, M, Cin = g.shape
    QG = min(M, 128)
    Cout = layers[-1][0].shape[1]
    wargs = []
    wspecs = []
    for (W, b) in layers:
        wargs += [W, b]
        wspecs += [
            pl.BlockSpec(W.shape, lambda b_, q_: (0, 0)),
            pl.BlockSpec(b.shape, lambda b_, q_: (0,)),
        ]
    out = pl.pallas_call(
        functools.partial(_sa_mlp_kernel, nsample=ns),
        out_shape=jax.ShapeDtypeStruct((B, M, Cout), jnp.float32),
        grid=(B, M // QG),
        in_specs=[pl.BlockSpec((1, ns, QG, Cin), lambda b_, q_: (b_, 0, q_, 0))] + wspecs,
        out_specs=pl.BlockSpec((1, QG, Cout), lambda b_, q_: (b_, q_, 0)),
    )(g, *wargs)
    return out


def _sa_stage(xyz_rows, feat, new_xyz_rows, q_planes, nsample, layers):
    B, N, _ = xyz_rows.shape
    M = q_planes.shape[1] * 128
    nn = _knn(xyz_rows, q_planes, nsample)
    table = jnp.concatenate([xyz_rows, feat], -1)
    g = _take_rows(table, nn)
    nxp = new_xyz_rows
    if M != new_xyz_rows.shape[1]:
        nxp = jnp.zeros((B, M, 3), jnp.float32).at[:, : new_xyz_rows.shape[1]].set(new_xyz_rows)
    g = g.at[..., :3].add(-nxp[:, None, :, :])
    return _sa_mlp_max(g, layers)


def _fp_kernel(ref_ref, q_ref, f2_ref, f1_ref, *rest):
    w_refs = rest[:-1]
    o_ref = rest[-1]
    n2 = ref_ref.shape[1]
    rx = ref_ref[0, :, 0:1]
    ry = ref_ref[0, :, 1:2]
    rz = ref_ref[0, :, 2:3]
    qx = q_ref[0, 0, 0, :].reshape(1, 128)
    qy = q_ref[0, 0, 1, :].reshape(1, 128)
    qz = q_ref[0, 0, 2, :].reshape(1, 128)
    d = (qx - rx) ** 2 + (qy - ry) ** 2 + (qz - rz) ** 2
    ri = jax.lax.broadcasted_iota(jnp.int32, (n2, 128), 0)
    big_i = jnp.int32(1 << 30)
    ms, idxs = [], []
    for j in range(3):
        m = jnp.min(d, axis=0, keepdims=True)
        idx = jnp.min(jnp.where(d == m, ri, big_i), axis=0, keepdims=True)
        ms.append(m)
        idxs.append(idx)
        d = jnp.where(ri == idx, jnp.float32(jnp.inf), d)
    ws = [1.0 / jnp.maximum(m, 1e-10) for m in ms]
    tot = ws[0] + ws[1] + ws[2]
    Wm = jnp.zeros((n2, 128), jnp.float32)
    for j in range(3):
        Wm = Wm + jnp.where(ri == idxs[j], ws[j] / tot, 0.0)
    interp = jax.lax.dot_general(
        Wm, f2_ref[0], (((0,), (0,)), ((), ())),
        precision=jax.lax.Precision.HIGHEST,
    )
    h = jnp.concatenate([interp, f1_ref[0]], axis=1)
    nl = len(w_refs) // 2
    for i in range(nl):
        W = w_refs[2 * i][...]
        b = w_refs[2 * i + 1][...]
        h = jnp.maximum(jnp.dot(h, W, precision=jax.lax.Precision.HIGHEST) + b, 0.0)
    o_ref[0] = h


def _fp_stage(ref_rows, q_planes, feat1, feat2, layers):
    B, n2, _ = ref_rows.shape
    QB = q_planes.shape[1]
    M = QB * 128
    C1 = feat1.shape[2]
    Cout = layers[-1][0].shape[1]
    wargs = []
    wspecs = []
    for (W, b) in layers:
        wargs += [W, b]
        wspecs += [
            pl.BlockSpec(W.shape, lambda b_, q_: (0, 0)),
            pl.BlockSpec(b.shape, lambda b_, q_: (0,)),
        ]
    out = pl.pallas_call(
        _fp_kernel,
        out_shape=jax.ShapeDtypeStruct((B, M, Cout), jnp.float32),
        grid=(B, QB),
        in_specs=[
            pl.BlockSpec((1, n2, 3), lambda b_, q_: (b_, 0, 0)),
            pl.BlockSpec((1, 1, 3, 128), lambda b_, q_: (b_, q_, 0, 0)),
            pl.BlockSpec((1, n2, feat2.shape[2]), lambda b_, q_: (b_, 0, 0)),
            pl.BlockSpec((1, 128, C1), lambda b_, q_: (b_, q_, 0)),
        ] + wspecs,
        out_specs=pl.BlockSpec((1, 128, Cout), lambda b_, q_: (b_, q_, 0)),
    )(ref_rows, q_planes, feat2, feat1, *wargs)
    return out


def _rows_to_planes(rows, pad_val=0.0):
    B, M, _ = rows.shape
    QB = max(1, (M + 127) // 128)
    p = jnp.transpose(rows, (0, 2, 1))
    if QB * 128 != M:
        p = jnp.concatenate(
            [p, jnp.full((B, 3, QB * 128 - M), pad_val, jnp.float32)], axis=2
        )
    return jnp.swapaxes(p.reshape(B, 3, QB, 128), 1, 2)


def _pad_rows(x, M):
    B, n, C = x.shape
    if n == M:
        return x
    return jnp.concatenate([x, jnp.zeros((B, M - n, C), x.dtype)], axis=1)


def kernel(pointcloud, params):
    B = pointcloud.shape[0]
    xyz = pointcloud[..., :3]
    feat = pointcloud[..., 3:]
    fps_planes = _fps_all_planes(xyz)
    nx_planes = [jnp.swapaxes(p, 1, 2) for p in fps_planes]
    nx_rows = []
    for p, npoint in zip(fps_planes, _NPOINTS):
        nx_rows.append(jnp.transpose(p.reshape(B, 3, -1), (0, 2, 1))[:, :npoint])

    l_xyz_rows = [xyz] + nx_rows
    l_feat = [feat]
    for i in range(4):
        nf = _sa_stage(
            l_xyz_rows[i], l_feat[i], nx_rows[i], nx_planes[i],
            _NSAMPLES[i], params["sa"][i],
        )
        l_feat.append(nf[:, : _NPOINTS[i]])

    xyz0_planes = _rows_to_planes(xyz)
    fp_q_planes = [xyz0_planes, nx_planes[0], nx_planes[1], nx_planes[2]]
    for i in range(-1, -5, -1):
        lev = 4 + i
        q_planes = fp_q_planes[lev]
        M = q_planes.shape[1] * 128
        feat1 = _pad_rows(l_feat[lev], M)
        ref_rows = l_xyz_rows[lev + 1]
        feat2 = l_feat[lev + 1]
        out = _fp_stage(ref_rows, q_planes, feat1, feat2, params["fp"][i])
        npts = l_xyz_rows[lev].shape[1]
        l_feat[lev] = out[:, :npts]

    h = _fc_head(l_feat[0], params["fc"])
    return jnp.transpose(h, (0, 2, 1))


# speedup vs baseline: 5.9756x; 1.5590x over previous
import functools
import jax
import jax.numpy as jnp
from jax import lax
from jax.experimental import pallas as pl
from jax.experimental.pallas import tpu as pltpu
from jax.experimental.pallas import tpu_sc as plsc


def _sc_gather(table, idx2d):
    V, D = table.shape
    nch = idx2d.shape[0]
    M = nch * 128
    info = plsc.get_sparse_core_info()
    NW = info.num_cores * info.num_subcores
    per_w = nch // NW
    mesh = plsc.VectorSubcoreMesh(core_axis_name="c", subcore_axis_name="s")

    @functools.partial(
        pl.kernel, mesh=mesh,
        compiler_params=pltpu.CompilerParams(use_tc_tiling_on_sc=False),
        out_type=jax.ShapeDtypeStruct((M, D), jnp.float32),
        scratch_types=[
            pltpu.VMEM((per_w, 128), jnp.int32),
            pltpu.VMEM((per_w * 128, D), jnp.float32),
            pltpu.SemaphoreType.DMA,
        ],
    )
    def k(table_hbm, idx_hbm, out_hbm, idx_v, rows_v, sem):
        wid = lax.axis_index("s") * info.num_cores + lax.axis_index("c")
        rowbase = wid * per_w
        pltpu.sync_copy(idx_hbm.at[pl.ds(rowbase, per_w)], idx_v)
        hs = [
            pltpu.async_copy(
                table_hbm.at[idx_v.at[j]], rows_v.at[pl.ds(j * 128, 128)], sem
            )
            for j in range(per_w)
        ]
        for h in hs:
            h.wait()
        pltpu.sync_copy(rows_v, out_hbm.at[pl.ds(rowbase * 128, per_w * 128)])

    return k(table, idx2d)

_NPOINTS = [1024, 256, 64, 16]
_NSAMPLES = [32, 32, 32, 32]


def _fc_head_kernel(x_ref, w1_ref, b1_ref, w2_ref, b2_ref, o_ref):
    h = jnp.maximum(x_ref[...] @ w1_ref[...] + b1_ref[...], 0.0)
    o_ref[...] = h @ w2_ref[...] + b2_ref[...]


def _fc_head(x, fc_params):
    (w1, b1), (w2, b2) = fc_params
    B, N, C = x.shape
    C2 = w2.shape[1]
    C2p = 128
    w2p = jnp.zeros((w2.shape[0], C2p), w2.dtype).at[:, :C2].set(w2)
    b2p = jnp.zeros((C2p,), b2.dtype).at[:C2].set(b2)
    xf = x.reshape(B * N, C)
    out = pl.pallas_call(
        _fc_head_kernel,
        out_shape=jax.ShapeDtypeStruct((B * N, C2p), jnp.float32),
        grid=(B * N // 2048,),
        in_specs=[
            pl.BlockSpec((2048, C), lambda i: (i, 0)),
            pl.BlockSpec((C, C), lambda i: (0, 0)),
            pl.BlockSpec((C,), lambda i: (0,)),
            pl.BlockSpec((C, C2p), lambda i: (0, 0)),
            pl.BlockSpec((C2p,), lambda i: (0,)),
        ],
        out_specs=pl.BlockSpec((2048, C2p), lambda i: (i, 0)),
    )(xf, w1, b1, w2p, b2p)
    return out.reshape(B, N, C2p)[:, :, :C2]


_SENT = 1e9


def _fps_level(src, valid_n, npoint, dst_ref, b):
    S = src[0].shape[0]
    Sp = dst_ref.shape[2]
    fi = (jax.lax.broadcasted_iota(jnp.int32, (S, 128), 0) * 128
          + jax.lax.broadcasted_iota(jnp.int32, (S, 128), 1))
    fq = (jax.lax.broadcasted_iota(jnp.int32, (Sp, 128), 0) * 128
          + jax.lax.broadcasted_iota(jnp.int32, (Sp, 128), 1))
    x, y, z = src
    dist0 = jnp.where(fi < valid_n, 1e10, -1.0).astype(jnp.float32)

    def body(step, carry):
        dist, far = carry
        m = fi == far
        cx = jnp.sum(jnp.where(m, x, 0.0))
        cy = jnp.sum(jnp.where(m, y, 0.0))
        cz = jnp.sum(jnp.where(m, z, 0.0))
        mq = fq == step
        dst_ref[b, 0] = jnp.where(mq, cx, dst_ref[b, 0])
        dst_ref[b, 1] = jnp.where(mq, cy, dst_ref[b, 1])
        dst_ref[b, 2] = jnp.where(mq, cz, dst_ref[b, 2])
        d = (x - cx) ** 2 + (y - cy) ** 2 + (z - cz) ** 2
        dist = jnp.minimum(dist, d)
        mx = jnp.max(dist)
        far = jnp.min(jnp.where(dist == mx, fi, jnp.int32(1 << 30)))
        return dist, far

    jax.lax.fori_loop(0, npoint, body, (dist0, jnp.int32(0)), unroll=False)


def _fps_all_kernel(xyz_ref, nx1_ref, nx2_ref, nx3_ref, nx4_ref):
    nx1_ref[...] = jnp.zeros(nx1_ref.shape, jnp.float32)
    nx2_ref[...] = jnp.zeros(nx2_ref.shape, jnp.float32)
    nx3_ref[...] = jnp.full(nx3_ref.shape, _SENT, jnp.float32)
    nx4_ref[...] = jnp.full(nx4_ref.shape, _SENT, jnp.float32)
    for b in range(xyz_ref.shape[0]):
        src0 = [xyz_ref[b, c] for c in range(3)]
        _fps_level(src0, 8192, 1024, nx1_ref, b)
        src1 = [nx1_ref[b, c] for c in range(3)]
        _fps_level(src1, 1024, 256, nx2_ref, b)
        src2 = [nx2_ref[b, c] for c in range(3)]
        _fps_level(src2, 256, 64, nx3_ref, b)
        src3 = [nx3_ref[b, c] for c in range(3)]
        _fps_level(src3, 64, 16, nx4_ref, b)


def _fps_all_planes(xyz):
    Bn, N, _ = xyz.shape
    xp = jnp.transpose(xyz, (0, 2, 1)).reshape(Bn, 3, N // 128, 128)
    return pl.pallas_call(
        _fps_all_kernel,
        out_shape=[
            jax.ShapeDtypeStruct((Bn, 3, 8, 128), jnp.float32),
            jax.ShapeDtypeStruct((Bn, 3, 2, 128), jnp.float32),
            jax.ShapeDtypeStruct((Bn, 3, 1, 128), jnp.float32),
            jax.ShapeDtypeStruct((Bn, 3, 1, 128), jnp.float32),
        ],
    )(xp)


def _take_rows(x, idx):
    bi = jnp.arange(x.shape[0]).reshape((x.shape[0],) + (1,) * (idx.ndim - 1))
    return x[bi, idx]


def _knn_kernel(ref_ref, q_ref, o_ref, *, k):
    N = ref_ref.shape[1]
    rx = ref_ref[0, :, 0:1]
    ry = ref_ref[0, :, 1:2]
    rz = ref_ref[0, :, 2:3]
    qx = q_ref[0, 0, 0, :].reshape(1, 128)
    qy = q_ref[0, 0, 1, :].reshape(1, 128)
    qz = q_ref[0, 0, 2, :].reshape(1, 128)
    d = (qx - rx) ** 2 + (qy - ry) ** 2 + (qz - rz) ** 2
    ri = jax.lax.broadcasted_iota(jnp.int32, (N, 128), 0)
    big_i = jnp.int32(1 << 30)
    for j in range(k):
        m = jnp.min(d, axis=0, keepdims=True)
        idx = jnp.min(jnp.where(d == m, ri, big_i), axis=0, keepdims=True)
        o_ref[0, 0, j, :] = idx.reshape(128)
        d = jnp.where(ri == idx, jnp.float32(jnp.inf), d)


def _knn(ref_rows, q_planes, k):
    B, N, _ = ref_rows.shape
    QB = q_planes.shape[1]
    out = pl.pallas_call(
        functools.partial(_knn_kernel, k=k),
        out_shape=jax.ShapeDtypeStruct((B, QB, k, 128), jnp.int32),
        grid=(B, QB),
        in_specs=[
            pl.BlockSpec((1, N, 3), lambda b, q: (b, 0, 0)),
            pl.BlockSpec((1, 1, 3, 128), lambda b, q: (b, q, 0, 0)),
        ],
        out_specs=pl.BlockSpec((1, 1, k, 128), lambda b, q: (b, q, 0, 0)),
    )(ref_rows, q_planes)
    return jnp.transpose(out, (0, 2, 1, 3)).reshape(B, k, QB * 128)


def _sa_mlp_kernel(g_ref, nx_ref, w1x_ref, *rest, nsample):
    w_refs = rest[:-1]
    o_ref = rest[-1]
    ns, QG, Dp = g_ref.shape[1], g_ref.shape[2], g_ref.shape[3]
    h = g_ref[0].reshape(ns * QG, Dp)
    nl = len(w_refs) // 2
    corr = jnp.dot(nx_ref[0], w1x_ref[...], precision=jax.lax.Precision.HIGHEST)
    for i in range(nl):
        W = w_refs[2 * i][...]
        b = w_refs[2 * i + 1][...]
        h = jnp.dot(h, W, precision=jax.lax.Precision.HIGHEST) + b
        if i == 0:
            C1 = h.shape[1]
            h = (h.reshape(ns, QG, C1) - corr[None]).reshape(ns * QG, C1)
        h = jnp.maximum(h, 0.0)
    Cout = h.shape[1]
    h = h.reshape(ns, QG, Cout)
    acc = h[0]
    for s in range(1, ns):
        acc = jnp.maximum(acc, h[s])
    o_ref[0] = acc


def _sa_mlp_max(g, nxp, layers):
    B, ns, M, Dp = g.shape
    QG = min(M, 128)
    Cout = layers[-1][0].shape[1]
    (W1, b1) = layers[0]
    Cin = W1.shape[0]
    W1p = W1 if Cin == Dp else jnp.concatenate(
        [W1, jnp.zeros((Dp - Cin, W1.shape[1]), jnp.float32)], axis=0)
    W1x = W1[:3]
    wargs = [W1p, b1]
    wspecs = [
        pl.BlockSpec(W1p.shape, lambda b_, q_: (0, 0)),
        pl.BlockSpec(b1.shape, lambda b_, q_: (0,)),
    ]
    for (W, b) in layers[1:]:
        wargs += [W, b]
        wspecs += [
            pl.BlockSpec(W.shape, lambda b_, q_: (0, 0)),
            pl.BlockSpec(b.shape, lambda b_, q_: (0,)),
        ]
    out = pl.pallas_call(
        functools.partial(_sa_mlp_kernel, nsample=ns),
        out_shape=jax.ShapeDtypeStruct((B, M, Cout), jnp.float32),
        grid=(B, M // QG),
        in_specs=[
            pl.BlockSpec((1, ns, QG, Dp), lambda b_, q_: (b_, 0, q_, 0)),
            pl.BlockSpec((1, QG, 3), lambda b_, q_: (b_, q_, 0)),
            pl.BlockSpec(W1x.shape, lambda b_, q_: (0, 0)),
        ] + wspecs,
        out_specs=pl.BlockSpec((1, QG, Cout), lambda b_, q_: (b_, q_, 0)),
    )(g, nxp, W1x, *wargs)
    return out


def _sa_stage(xyz_rows, feat, new_xyz_rows, q_planes, nsample, layers):
    B, N, _ = xyz_rows.shape
    M = q_planes.shape[1] * 128
    nn = _knn(xyz_rows, q_planes, nsample)
    C = 3 + feat.shape[2]
    Dp = ((C + 15) // 16) * 16
    table = jnp.concatenate([xyz_rows, feat], -1)
    if Dp != C:
        table = jnp.concatenate(
            [table, jnp.zeros((B, N, Dp - C), jnp.float32)], -1)
    table = table.reshape(B * N, Dp)
    idx = (nn + (jnp.arange(B, dtype=jnp.int32) * N)[:, None, None]).reshape(-1, 128)
    g = _sc_gather(table, idx).reshape(B, nsample, M, Dp)
    nxp = new_xyz_rows
    if M != new_xyz_rows.shape[1]:
        nxp = jnp.zeros((B, M, 3), jnp.float32).at[:, : new_xyz_rows.shape[1]].set(new_xyz_rows)
    return _sa_mlp_max(g, nxp, layers)


def _fp_kernel(ref_ref, q_ref, f2_ref, f1_ref, *rest):
    w_refs = rest[:-1]
    o_ref = rest[-1]
    n2 = ref_ref.shape[1]
    rx = ref_ref[0, :, 0:1]
    ry = ref_ref[0, :, 1:2]
    rz = ref_ref[0, :, 2:3]
    qx = q_ref[0, 0, 0, :].reshape(1, 128)
    qy = q_ref[0, 0, 1, :].reshape(1, 128)
    qz = q_ref[0, 0, 2, :].reshape(1, 128)
    d = (qx - rx) ** 2 + (qy - ry) ** 2 + (qz - rz) ** 2
    ri = jax.lax.broadcasted_iota(jnp.int32, (n2, 128), 0)
    big_i = jnp.int32(1 << 30)
    ms, idxs = [], []
    for j in range(3):
        m = jnp.min(d, axis=0, keepdims=True)
        idx = jnp.min(jnp.where(d == m, ri, big_i), axis=0, keepdims=True)
        ms.append(m)
        idxs.append(idx)
        d = jnp.where(ri == idx, jnp.float32(jnp.inf), d)
    ws = [1.0 / jnp.maximum(m, 1e-10) for m in ms]
    tot = ws[0] + ws[1] + ws[2]
    Wm = jnp.zeros((n2, 128), jnp.float32)
    for j in range(3):
        Wm = Wm + jnp.where(ri == idxs[j], ws[j] / tot, 0.0)
    interp = jax.lax.dot_general(
        Wm, f2_ref[0], (((0,), (0,)), ((), ())),
        precision=jax.lax.Precision.HIGHEST,
    )
    h = jnp.concatenate([interp, f1_ref[0]], axis=1)
    nl = len(w_refs) // 2
    for i in range(nl):
        W = w_refs[2 * i][...]
        b = w_refs[2 * i + 1][...]
        h = jnp.maximum(jnp.dot(h, W, precision=jax.lax.Precision.HIGHEST) + b, 0.0)
    o_ref[0] = h


def _fp_stage(ref_rows, q_planes, feat1, feat2, layers):
    B, n2, _ = ref_rows.shape
    QB = q_planes.shape[1]
    M = QB * 128
    C1 = feat1.shape[2]
    Cout = layers[-1][0].shape[1]
    wargs = []
    wspecs = []
    for (W, b) in layers:
        wargs += [W, b]
        wspecs += [
            pl.BlockSpec(W.shape, lambda b_, q_: (0, 0)),
            pl.BlockSpec(b.shape, lambda b_, q_: (0,)),
        ]
    out = pl.pallas_call(
        _fp_kernel,
        out_shape=jax.ShapeDtypeStruct((B, M, Cout), jnp.float32),
        grid=(B, QB),
        in_specs=[
            pl.BlockSpec((1, n2, 3), lambda b_, q_: (b_, 0, 0)),
            pl.BlockSpec((1, 1, 3, 128), lambda b_, q_: (b_, q_, 0, 0)),
            pl.BlockSpec((1, n2, feat2.shape[2]), lambda b_, q_: (b_, 0, 0)),
            pl.BlockSpec((1, 128, C1), lambda b_, q_: (b_, q_, 0)),
        ] + wspecs,
        out_specs=pl.BlockSpec((1, 128, Cout), lambda b_, q_: (b_, q_, 0)),
    )(ref_rows, q_planes, feat2, feat1, *wargs)
    return out


def _rows_to_planes(rows, pad_val=0.0):
    B, M, _ = rows.shape
    QB = max(1, (M + 127) // 128)
    p = jnp.transpose(rows, (0, 2, 1))
    if QB * 128 != M:
        p = jnp.concatenate(
            [p, jnp.full((B, 3, QB * 128 - M), pad_val, jnp.float32)], axis=2
        )
    return jnp.swapaxes(p.reshape(B, 3, QB, 128), 1, 2)


def _pad_rows(x, M):
    B, n, C = x.shape
    if n == M:
        return x
    return jnp.concatenate([x, jnp.zeros((B, M - n, C), x.dtype)], axis=1)


def kernel(pointcloud, params):
    B = pointcloud.shape[0]
    xyz = pointcloud[..., :3]
    feat = pointcloud[..., 3:]
    fps_planes = _fps_all_planes(xyz)
    nx_planes = [jnp.swapaxes(p, 1, 2) for p in fps_planes]
    nx_rows = []
    for p, npoint in zip(fps_planes, _NPOINTS):
        nx_rows.append(jnp.transpose(p.reshape(B, 3, -1), (0, 2, 1))[:, :npoint])

    l_xyz_rows = [xyz] + nx_rows
    l_feat = [feat]
    for i in range(4):
        nf = _sa_stage(
            l_xyz_rows[i], l_feat[i], nx_rows[i], nx_planes[i],
            _NSAMPLES[i], params["sa"][i],
        )
        l_feat.append(nf[:, : _NPOINTS[i]])

    xyz0_planes = _rows_to_planes(xyz)
    fp_q_planes = [xyz0_planes, nx_planes[0], nx_planes[1], nx_planes[2]]
    for i in range(-1, -5, -1):
        lev = 4 + i
        q_planes = fp_q_planes[lev]
        M = q_planes.shape[1] * 128
        feat1 = _pad_rows(l_feat[lev], M)
        ref_rows = l_xyz_rows[lev + 1]
        feat2 = l_feat[lev + 1]
        out = _fp_stage(ref_rows, q_planes, feat1, feat2, params["fp"][i])
        npts = l_xyz_rows[lev].shape[1]
        l_feat[lev] = out[:, :npts]

    h = _fc_head(l_feat[0], params["fc"])
    return jnp.transpose(h, (0, 2, 1))


# speedup vs baseline: 6.3162x; 1.0570x over previous
import functools
import jax
import jax.numpy as jnp
from jax import lax
from jax.experimental import pallas as pl
from jax.experimental.pallas import tpu as pltpu
from jax.experimental.pallas import tpu_sc as plsc


def _sc_gather(table, idx2d):
    V, D = table.shape
    nch = idx2d.shape[0]
    M = nch * 128
    info = plsc.get_sparse_core_info()
    NW = info.num_cores * info.num_subcores
    per_w = nch // NW
    mesh = plsc.VectorSubcoreMesh(core_axis_name="c", subcore_axis_name="s")

    @functools.partial(
        pl.kernel, mesh=mesh,
        compiler_params=pltpu.CompilerParams(use_tc_tiling_on_sc=False),
        out_type=jax.ShapeDtypeStruct((M, D), jnp.float32),
        scratch_types=[
            pltpu.VMEM((per_w, 128), jnp.int32),
            pltpu.VMEM((per_w * 128, D), jnp.float32),
            pltpu.SemaphoreType.DMA,
        ],
    )
    def k(table_hbm, idx_hbm, out_hbm, idx_v, rows_v, sem):
        wid = lax.axis_index("s") * info.num_cores + lax.axis_index("c")
        rowbase = wid * per_w
        pltpu.sync_copy(idx_hbm.at[pl.ds(rowbase, per_w)], idx_v)
        hs = [
            pltpu.async_copy(
                table_hbm.at[idx_v.at[j]], rows_v.at[pl.ds(j * 128, 128)], sem
            )
            for j in range(per_w)
        ]
        for h in hs:
            h.wait()
        pltpu.sync_copy(rows_v, out_hbm.at[pl.ds(rowbase * 128, per_w * 128)])

    return k(table, idx2d)

_NPOINTS = [1024, 256, 64, 16]
_NSAMPLES = [32, 32, 32, 32]


def _fc_head_kernel(x_ref, w1_ref, b1_ref, w2_ref, b2_ref, o_ref):
    h = jnp.maximum(x_ref[...] @ w1_ref[...] + b1_ref[...], 0.0)
    o_ref[...] = h @ w2_ref[...] + b2_ref[...]


def _fc_head(x, fc_params):
    (w1, b1), (w2, b2) = fc_params
    B, N, C = x.shape
    C2 = w2.shape[1]
    C2p = 128
    w2p = jnp.zeros((w2.shape[0], C2p), w2.dtype).at[:, :C2].set(w2)
    b2p = jnp.zeros((C2p,), b2.dtype).at[:C2].set(b2)
    xf = x.reshape(B * N, C)
    out = pl.pallas_call(
        _fc_head_kernel,
        out_shape=jax.ShapeDtypeStruct((B * N, C2p), jnp.float32),
        grid=(B * N // 2048,),
        in_specs=[
            pl.BlockSpec((2048, C), lambda i: (i, 0)),
            pl.BlockSpec((C, C), lambda i: (0, 0)),
            pl.BlockSpec((C,), lambda i: (0,)),
            pl.BlockSpec((C, C2p), lambda i: (0, 0)),
            pl.BlockSpec((C2p,), lambda i: (0,)),
        ],
        out_specs=pl.BlockSpec((2048, C2p), lambda i: (i, 0)),
    )(xf, w1, b1, w2p, b2p)
    return out.reshape(B, N, C2p)[:, :, :C2]


_SENT = 1e9


def _fps_level(src_ref, valid_n, npoint, dst_ref):
    B = src_ref.shape[0]
    S = src_ref.shape[2]
    Sp = dst_ref.shape[2]
    fi = (jax.lax.broadcasted_iota(jnp.int32, (S, 128), 0) * 128
          + jax.lax.broadcasted_iota(jnp.int32, (S, 128), 1))
    fq = (jax.lax.broadcasted_iota(jnp.int32, (Sp, 128), 0) * 128
          + jax.lax.broadcasted_iota(jnp.int32, (Sp, 128), 1))
    dist0 = jnp.where(fi < valid_n, 1e10, -1.0).astype(jnp.float32)

    li = jax.lax.broadcasted_iota(jnp.int32, (1, 128), 1)

    def body(step, carry):
        outs = []
        mq = fq == step
        for b in range(B):
            dist, far = carry[2 * b], carry[2 * b + 1]
            r = far >> 7
            c = far & 127
            lm = li == c
            cx = jnp.sum(jnp.where(lm, src_ref[b, 0, pl.ds(r, 1)], 0.0))
            cy = jnp.sum(jnp.where(lm, src_ref[b, 1, pl.ds(r, 1)], 0.0))
            cz = jnp.sum(jnp.where(lm, src_ref[b, 2, pl.ds(r, 1)], 0.0))
            dst_ref[b, 0] = jnp.where(mq, cx, dst_ref[b, 0])
            dst_ref[b, 1] = jnp.where(mq, cy, dst_ref[b, 1])
            dst_ref[b, 2] = jnp.where(mq, cz, dst_ref[b, 2])
            d = ((src_ref[b, 0] - cx) ** 2 + (src_ref[b, 1] - cy) ** 2
                 + (src_ref[b, 2] - cz) ** 2)
            dist = jnp.minimum(dist, d)
            mx = jnp.max(dist)
            far = jnp.min(jnp.where(dist == mx, fi, jnp.int32(1 << 30)))
            outs += [dist, far]
        return tuple(outs)

    init = ()
    for b in range(B):
        init += (dist0, jnp.int32(0))
    jax.lax.fori_loop(0, npoint, body, init, unroll=False)


def _fps_all_kernel(xyz_ref, nx1_ref, nx2_ref, nx3_ref, nx4_ref):
    nx1_ref[...] = jnp.zeros(nx1_ref.shape, jnp.float32)
    nx2_ref[...] = jnp.zeros(nx2_ref.shape, jnp.float32)
    nx3_ref[...] = jnp.full(nx3_ref.shape, _SENT, jnp.float32)
    nx4_ref[...] = jnp.full(nx4_ref.shape, _SENT, jnp.float32)
    _fps_level(xyz_ref, 8192, 1024, nx1_ref)
    _fps_level(nx1_ref, 1024, 256, nx2_ref)
    _fps_level(nx2_ref, 256, 64, nx3_ref)
    _fps_level(nx3_ref, 64, 16, nx4_ref)


def _fps_all_planes(xyz):
    Bn, N, _ = xyz.shape
    xp = jnp.transpose(xyz, (0, 2, 1)).reshape(Bn, 3, N // 128, 128)
    return pl.pallas_call(
        _fps_all_kernel,
        out_shape=[
            jax.ShapeDtypeStruct((Bn, 3, 8, 128), jnp.float32),
            jax.ShapeDtypeStruct((Bn, 3, 2, 128), jnp.float32),
            jax.ShapeDtypeStruct((Bn, 3, 1, 128), jnp.float32),
            jax.ShapeDtypeStruct((Bn, 3, 1, 128), jnp.float32),
        ],
    )(xp)


def _take_rows(x, idx):
    bi = jnp.arange(x.shape[0]).reshape((x.shape[0],) + (1,) * (idx.ndim - 1))
    return x[bi, idx]


def _knn_kernel(ref_ref, q_ref, o_ref, *, k):
    N = ref_ref.shape[1]
    rx = ref_ref[0, :, 0:1]
    ry = ref_ref[0, :, 1:2]
    rz = ref_ref[0, :, 2:3]
    qx = q_ref[0, 0, 0, :].reshape(1, 128)
    qy = q_ref[0, 0, 1, :].reshape(1, 128)
    qz = q_ref[0, 0, 2, :].reshape(1, 128)
    d = (qx - rx) ** 2 + (qy - ry) ** 2 + (qz - rz) ** 2
    ri = jax.lax.broadcasted_iota(jnp.int32, (N, 128), 0)
    big_i = jnp.int32(1 << 30)
    for j in range(k):
        m = jnp.min(d, axis=0, keepdims=True)
        idx = jnp.min(jnp.where(d == m, ri, big_i), axis=0, keepdims=True)
        o_ref[0, 0, j, :] = idx.reshape(128)
        d = jnp.where(ri == idx, jnp.float32(jnp.inf), d)


def _knn(ref_rows, q_planes, k):
    B, N, _ = ref_rows.shape
    QB = q_planes.shape[1]
    out = pl.pallas_call(
        functools.partial(_knn_kernel, k=k),
        out_shape=jax.ShapeDtypeStruct((B, QB, k, 128), jnp.int32),
        grid=(B, QB),
        in_specs=[
            pl.BlockSpec((1, N, 3), lambda b, q: (b, 0, 0)),
            pl.BlockSpec((1, 1, 3, 128), lambda b, q: (b, q, 0, 0)),
        ],
        out_specs=pl.BlockSpec((1, 1, k, 128), lambda b, q: (b, q, 0, 0)),
    )(ref_rows, q_planes)
    return jnp.transpose(out, (0, 2, 1, 3)).reshape(B, k, QB * 128)


def _sa_mlp_kernel(g_ref, nx_ref, w1x_ref, *rest, nsample):
    w_refs = rest[:-1]
    o_ref = rest[-1]
    ns, QG, Dp = g_ref.shape[1], g_ref.shape[2], g_ref.shape[3]
    h = g_ref[0].reshape(ns * QG, Dp)
    nl = len(w_refs) // 2
    corr = jnp.dot(nx_ref[0], w1x_ref[...], precision=jax.lax.Precision.HIGHEST)
    for i in range(nl):
        W = w_refs[2 * i][...]
        b = w_refs[2 * i + 1][...]
        h = jnp.dot(h, W, precision=jax.lax.Precision.HIGHEST) + b
        if i == 0:
            C1 = h.shape[1]
            h = (h.reshape(ns, QG, C1) - corr[None]).reshape(ns * QG, C1)
        h = jnp.maximum(h, 0.0)
    Cout = h.shape[1]
    h = h.reshape(ns, QG, Cout)
    acc = h[0]
    for s in range(1, ns):
        acc = jnp.maximum(acc, h[s])
    o_ref[0] = acc


def _sa_mlp_max(g, nxp, layers):
    B, ns, M, Dp = g.shape
    QG = min(M, 128)
    Cout = layers[-1][0].shape[1]
    (W1, b1) = layers[0]
    Cin = W1.shape[0]
    W1p = W1 if Cin == Dp else jnp.concatenate(
        [W1, jnp.zeros((Dp - Cin, W1.shape[1]), jnp.float32)], axis=0)
    W1x = W1[:3]
    wargs = [W1p, b1]
    wspecs = [
        pl.BlockSpec(W1p.shape, lambda b_, q_: (0, 0)),
        pl.BlockSpec(b1.shape, lambda b_, q_: (0,)),
    ]
    for (W, b) in layers[1:]:
        wargs += [W, b]
        wspecs += [
            pl.BlockSpec(W.shape, lambda b_, q_: (0, 0)),
            pl.BlockSpec(b.shape, lambda b_, q_: (0,)),
        ]
    out = pl.pallas_call(
        functools.partial(_sa_mlp_kernel, nsample=ns),
        out_shape=jax.ShapeDtypeStruct((B, M, Cout), jnp.float32),
        grid=(B, M // QG),
        in_specs=[
            pl.BlockSpec((1, ns, QG, Dp), lambda b_, q_: (b_, 0, q_, 0)),
            pl.BlockSpec((1, QG, 3), lambda b_, q_: (b_, q_, 0)),
            pl.BlockSpec(W1x.shape, lambda b_, q_: (0, 0)),
        ] + wspecs,
        out_specs=pl.BlockSpec((1, QG, Cout), lambda b_, q_: (b_, q_, 0)),
    )(g, nxp, W1x, *wargs)
    return out


def _sa_stage(xyz_rows, feat, new_xyz_rows, q_planes, nsample, layers):
    B, N, _ = xyz_rows.shape
    M = q_planes.shape[1] * 128
    nn = _knn(xyz_rows, q_planes, nsample)
    C = 3 + feat.shape[2]
    Dp = ((C + 15) // 16) * 16
    table = jnp.concatenate([xyz_rows, feat], -1)
    if Dp != C:
        table = jnp.concatenate(
            [table, jnp.zeros((B, N, Dp - C), jnp.float32)], -1)
    table = table.reshape(B * N, Dp)
    idx = (nn + (jnp.arange(B, dtype=jnp.int32) * N)[:, None, None]).reshape(-1, 128)
    g = _sc_gather(table, idx).reshape(B, nsample, M, Dp)
    nxp = new_xyz_rows
    if M != new_xyz_rows.shape[1]:
        nxp = jnp.zeros((B, M, 3), jnp.float32).at[:, : new_xyz_rows.shape[1]].set(new_xyz_rows)
    return _sa_mlp_max(g, nxp, layers)


def _fp_kernel(ref_ref, q_ref, f2_ref, f1_ref, *rest):
    w_refs = rest[:-1]
    o_ref = rest[-1]
    n2 = ref_ref.shape[1]
    rx = ref_ref[0, :, 0:1]
    ry = ref_ref[0, :, 1:2]
    rz = ref_ref[0, :, 2:3]
    qx = q_ref[0, 0, 0, :].reshape(1, 128)
    qy = q_ref[0, 0, 1, :].reshape(1, 128)
    qz = q_ref[0, 0, 2, :].reshape(1, 128)
    d = (qx - rx) ** 2 + (qy - ry) ** 2 + (qz - rz) ** 2
    ri = jax.lax.broadcasted_iota(jnp.int32, (n2, 128), 0)
    big_i = jnp.int32(1 << 30)
    ms, idxs = [], []
    for j in range(3):
        m = jnp.min(d, axis=0, keepdims=True)
        idx = jnp.min(jnp.where(d == m, ri, big_i), axis=0, keepdims=True)
        ms.append(m)
        idxs.append(idx)
        d = jnp.where(ri == idx, jnp.float32(jnp.inf), d)
    ws = [1.0 / jnp.maximum(m, 1e-10) for m in ms]
    tot = ws[0] + ws[1] + ws[2]
    Wm = jnp.zeros((n2, 128), jnp.float32)
    for j in range(3):
        Wm = Wm + jnp.where(ri == idxs[j], ws[j] / tot, 0.0)
    interp = jax.lax.dot_general(
        Wm, f2_ref[0], (((0,), (0,)), ((), ())),
        precision=jax.lax.Precision.HIGHEST,
    )
    h = jnp.concatenate([interp, f1_ref[0]], axis=1)
    nl = len(w_refs) // 2
    for i in range(nl):
        W = w_refs[2 * i][...]
        b = w_refs[2 * i + 1][...]
        h = jnp.maximum(jnp.dot(h, W, precision=jax.lax.Precision.HIGHEST) + b, 0.0)
    o_ref[0] = h


def _fp_stage(ref_rows, q_planes, feat1, feat2, layers):
    B, n2, _ = ref_rows.shape
    QB = q_planes.shape[1]
    M = QB * 128
    C1 = feat1.shape[2]
    Cout = layers[-1][0].shape[1]
    wargs = []
    wspecs = []
    for (W, b) in layers:
        wargs += [W, b]
        wspecs += [
            pl.BlockSpec(W.shape, lambda b_, q_: (0, 0)),
            pl.BlockSpec(b.shape, lambda b_, q_: (0,)),
        ]
    out = pl.pallas_call(
        _fp_kernel,
        out_shape=jax.ShapeDtypeStruct((B, M, Cout), jnp.float32),
        grid=(B, QB),
        in_specs=[
            pl.BlockSpec((1, n2, 3), lambda b_, q_: (b_, 0, 0)),
            pl.BlockSpec((1, 1, 3, 128), lambda b_, q_: (b_, q_, 0, 0)),
            pl.BlockSpec((1, n2, feat2.shape[2]), lambda b_, q_: (b_, 0, 0)),
            pl.BlockSpec((1, 128, C1), lambda b_, q_: (b_, q_, 0)),
        ] + wspecs,
        out_specs=pl.BlockSpec((1, 128, Cout), lambda b_, q_: (b_, q_, 0)),
    )(ref_rows, q_planes, feat2, feat1, *wargs)
    return out


def _rows_to_planes(rows, pad_val=0.0):
    B, M, _ = rows.shape
    QB = max(1, (M + 127) // 128)
    p = jnp.transpose(rows, (0, 2, 1))
    if QB * 128 != M:
        p = jnp.concatenate(
            [p, jnp.full((B, 3, QB * 128 - M), pad_val, jnp.float32)], axis=2
        )
    return jnp.swapaxes(p.reshape(B, 3, QB, 128), 1, 2)


def _pad_rows(x, M):
    B, n, C = x.shape
    if n == M:
        return x
    return jnp.concatenate([x, jnp.zeros((B, M - n, C), x.dtype)], axis=1)


def kernel(pointcloud, params):
    B = pointcloud.shape[0]
    xyz = pointcloud[..., :3]
    feat = pointcloud[..., 3:]
    fps_planes = _fps_all_planes(xyz)
    nx_planes = [jnp.swapaxes(p, 1, 2) for p in fps_planes]
    nx_rows = []
    for p, npoint in zip(fps_planes, _NPOINTS):
        nx_rows.append(jnp.transpose(p.reshape(B, 3, -1), (0, 2, 1))[:, :npoint])

    l_xyz_rows = [xyz] + nx_rows
    l_feat = [feat]
    for i in range(4):
        nf = _sa_stage(
            l_xyz_rows[i], l_feat[i], nx_rows[i], nx_planes[i],
            _NSAMPLES[i], params["sa"][i],
        )
        l_feat.append(nf[:, : _NPOINTS[i]])

    xyz0_planes = _rows_to_planes(xyz)
    fp_q_planes = [xyz0_planes, nx_planes[0], nx_planes[1], nx_planes[2]]
    for i in range(-1, -5, -1):
        lev = 4 + i
        q_planes = fp_q_planes[lev]
        M = q_planes.shape[1] * 128
        feat1 = _pad_rows(l_feat[lev], M)
        ref_rows = l_xyz_rows[lev + 1]
        feat2 = l_feat[lev + 1]
        out = _fp_stage(ref_rows, q_planes, feat1, feat2, params["fp"][i])
        npts = l_xyz_rows[lev].shape[1]
        l_feat[lev] = out[:, :npts]

    h = _fc_head(l_feat[0], params["fc"])
    return jnp.transpose(h, (0, 2, 1))
